# baseline (speedup 1.0000x reference)
import functools

import jax
import jax.numpy as jnp
from jax import lax
from jax.experimental import pallas as pl
from jax.experimental.pallas import tpu as pltpu
from jax.experimental.pallas import tpu_sc as plsc

_N = 128
_L = 20
_B = 64
_DK = 512
_DV = 512
_TOPK = 8

_NBLK = 32
_GRID = _N // _NBLK

_ROWS = _TOPK * _B * _L
_RLANE = 128
_NW = 32
_RPW = _ROWS // _NW
_CH = 80
_NCH = _RPW // _CH
_NBUF = 3


def _topk_body(q_ref, k_ref, w_ref, rows_ref):
    q = q_ref[...]
    kb = k_ref[...].reshape(_N * _B, _DK)
    s_full = lax.dot_general(
        kb, q, (((1,), (1,)), ((), ())),
        precision=lax.Precision.DEFAULT,
        preferred_element_type=jnp.float32)
    s3 = s_full.reshape(_N, _B, _B)
    diag = (lax.broadcasted_iota(jnp.int32, (_N, _B, _B), 1) ==
            lax.broadcasted_iota(jnp.int32, (_N, _B, _B), 2))
    s = jnp.sum(jnp.where(diag, s3, 0.0), axis=2)
    scores = s / jnp.sqrt(jnp.float32(_DK))
    m = jnp.max(scores, axis=0, keepdims=True)
    e = jnp.exp(scores - m)
    att = e / jnp.sum(e, axis=0, keepdims=True)
    iota = lax.broadcasted_iota(jnp.int32, (_N, _B), 0)
    cur = att
    idxs = []
    vals = []
    for _ in range(_TOPK):
        mv = jnp.max(cur, axis=0, keepdims=True)
        mi = jnp.min(jnp.where(cur == mv, iota, _N), axis=0, keepdims=True)
        idxs.append(mi)
        vals.append(mv)
        cur = jnp.where(iota == mi, -1.0, cur)
    idx = jnp.concatenate(idxs, axis=0)
    w = jnp.concatenate(vals, axis=0)
    wm = jnp.max(w, axis=0, keepdims=True)
    we = jnp.exp(w - wm)
    w_ref[...] = we / jnp.sum(we, axis=0, keepdims=True)
    idx2 = jnp.concatenate([idx, idx], axis=1)
    idx3 = jnp.broadcast_to(idx2[:, None, :], (_TOPK, 10, 2 * _B))
    idxr = idx3.reshape(_ROWS // _RLANE, _RLANE)
    r_iota = lax.broadcasted_iota(jnp.int32, (_ROWS // _RLANE, _RLANE), 0)
    s_iota = lax.broadcasted_iota(jnp.int32, (_ROWS // _RLANE, _RLANE), 1)
    l_arr = (2 * r_iota + s_iota // _B) % _L
    b_arr = s_iota % _B
    rows_ref[...] = idxr * (_L * _B) + l_arr * _B + b_arr


def _topk_call(q, keys):
    return pl.pallas_call(
        _topk_body,
        in_specs=[
            pl.BlockSpec((_B, _DK), lambda: (0, 0)),
            pl.BlockSpec((_N, _B, _DK), lambda: (0, 0, 0)),
        ],
        out_specs=[
            pl.BlockSpec((_TOPK, _B), lambda: (0, 0)),
            pl.BlockSpec((_ROWS // _RLANE, _RLANE), lambda: (0, 0)),
        ],
        out_shape=[
            jax.ShapeDtypeStruct((_TOPK, _B), jnp.float32),
            jax.ShapeDtypeStruct((_ROWS // _RLANE, _RLANE), jnp.int32),
        ],
    )(q, keys)


def _gather_body(rows_hbm, table_hbm, out_hbm, idx_v,
                 buf0, buf1, buf2, g0, g1, g2, s0, s1, s2):
    wid = lax.axis_index("s") * 2 + lax.axis_index("c")
    pltpu.sync_copy(rows_hbm.at[pl.ds(wid * _RPW, _RPW)], idx_v)
    bufs = (buf0, buf1, buf2)
    gsems = (g0, g1, g2)
    ssems = (s0, s1, s2)

    def gather(c):
        return pltpu.async_copy(
            table_hbm.at[idx_v.at[pl.ds(c * _CH, _CH)]],
            bufs[c % _NBUF], gsems[c % _NBUF])

    def scatter(c):
        return pltpu.async_copy(
            bufs[c % _NBUF],
            out_hbm.at[pl.ds(wid * _RPW + c * _CH, _CH)], ssems[c % _NBUF])

    handles = {}
    for c in range(min(_NBUF, _NCH)):
        handles[c] = gather(c)
    scat = {}
    for c in range(_NCH):
        handles[c].wait()
        scat[c] = scatter(c)
        nxt = c + _NBUF
        if nxt < _NCH:
            scat[c].wait()
            handles[nxt] = gather(nxt)
    for c in range(max(0, _NCH - _NBUF), _NCH):
        scat[c].wait()


def _gather_call(rows, table):
    mesh = plsc.VectorSubcoreMesh(core_axis_name="c", subcore_axis_name="s")
    f = functools.partial(
        pl.kernel,
        mesh=mesh,
        out_type=jax.ShapeDtypeStruct((_ROWS, _DV), jnp.float32),
        scratch_types=[
            pltpu.VMEM((_RPW,), jnp.int32),
            pltpu.VMEM((_CH, _DV), jnp.float32),
            pltpu.VMEM((_CH, _DV), jnp.float32),
            pltpu.VMEM((_CH, _DV), jnp.float32),
            pltpu.SemaphoreType.DMA,
            pltpu.SemaphoreType.DMA,
            pltpu.SemaphoreType.DMA,
            pltpu.SemaphoreType.DMA,
            pltpu.SemaphoreType.DMA,
            pltpu.SemaphoreType.DMA,
        ],
    )(_gather_body)
    return f(rows, table)


def kernel(query, keys, values):
    q = query.reshape(_B, _DK)
    w, rows = _topk_call(q, keys)
    rows = rows.reshape(_ROWS)
    table = values.reshape(_N * _L * _B, _DV)
    out = _gather_call(rows, table)
    topk_weights = jnp.transpose(w).reshape(_B, 1, _TOPK)
    outputs = jnp.swapaxes(out.reshape(_TOPK, _L, _B, _DV), 1, 2)
    return (topk_weights, outputs)


# speedup vs baseline: 1.0803x; 1.0803x over previous
import functools

import jax
import jax.numpy as jnp
from jax import lax
from jax.experimental import pallas as pl
from jax.experimental.pallas import tpu as pltpu
from jax.experimental.pallas import tpu_sc as plsc

_N = 128
_L = 20
_B = 64
_DK = 512
_DV = 512
_TOPK = 8

_NBLK = 32
_GRID = _N // _NBLK

_ROWS = _TOPK * _B * _L
_RLANE = 128
_NW = 32
_RPW = _ROWS // _NW
_CH = 80
_NCH = _RPW // _CH
_NBUF = 3


def _topk_body(q_hbm, keys_hbm, w_ref, rows_ref, q_v, s_ref,
               buf0, buf1, buf2, buf3, qsem, sem0, sem1, sem2, sem3):
    qcp = pltpu.make_async_copy(q_hbm, q_v, qsem)
    qcp.start()
    bufs = (buf0, buf1, buf2, buf3)
    sems = (sem0, sem1, sem2, sem3)

    def blk_copy(c):
        return pltpu.make_async_copy(
            keys_hbm.at[pl.ds(c * _NBLK, _NBLK)], bufs[c], sems[c])

    for c in range(_GRID):
        blk_copy(c).start()
    qcp.wait()
    q = q_v[...]
    for c in range(_GRID):
        blk_copy(c).wait()
        kb = bufs[c][...].reshape(_NBLK * _B, _DK)
        s_full = lax.dot_general(
            kb, q, (((1,), (1,)), ((), ())),
            precision=lax.Precision.DEFAULT,
            preferred_element_type=jnp.float32)
        s3 = s_full.reshape(_NBLK, _B, _B)
        diag = (lax.broadcasted_iota(jnp.int32, (_NBLK, _B, _B), 1) ==
                lax.broadcasted_iota(jnp.int32, (_NBLK, _B, _B), 2))
        s = jnp.sum(jnp.where(diag, s3, 0.0), axis=2)
        s_ref[pl.ds(c * _NBLK, _NBLK), :] = s / jnp.sqrt(jnp.float32(_DK))

    scores = s_ref[...]
    m = jnp.max(scores, axis=0, keepdims=True)
    e = jnp.exp(scores - m)
    att = e / jnp.sum(e, axis=0, keepdims=True)
    iota = lax.broadcasted_iota(jnp.int32, (_N, _B), 0)
    cur = att
    idxs = []
    vals = []
    for _ in range(_TOPK):
        mv = jnp.max(cur, axis=0, keepdims=True)
        mi = jnp.min(jnp.where(cur == mv, iota, _N), axis=0, keepdims=True)
        idxs.append(mi)
        vals.append(mv)
        cur = jnp.where(iota == mi, -1.0, cur)
    idx = jnp.concatenate(idxs, axis=0)
    w = jnp.concatenate(vals, axis=0)
    wm = jnp.max(w, axis=0, keepdims=True)
    we = jnp.exp(w - wm)
    w_ref[...] = we / jnp.sum(we, axis=0, keepdims=True)
    idx2 = jnp.concatenate([idx, idx], axis=1)
    idx3 = jnp.broadcast_to(idx2[:, None, :], (_TOPK, 10, 2 * _B))
    idxr = idx3.reshape(_ROWS // _RLANE, _RLANE)
    r_iota = lax.broadcasted_iota(jnp.int32, (_ROWS // _RLANE, _RLANE), 0)
    s_iota = lax.broadcasted_iota(jnp.int32, (_ROWS // _RLANE, _RLANE), 1)
    l_arr = (2 * r_iota + s_iota // _B) % _L
    b_arr = s_iota % _B
    rows_ref[...] = idxr * (_L * _B) + l_arr * _B + b_arr


def _topk_call(q, keys):
    return pl.pallas_call(
        _topk_body,
        in_specs=[
            pl.BlockSpec(memory_space=pl.ANY),
            pl.BlockSpec(memory_space=pl.ANY),
        ],
        out_specs=[
            pl.BlockSpec((_TOPK, _B), lambda: (0, 0)),
            pl.BlockSpec((_ROWS // _RLANE, _RLANE), lambda: (0, 0)),
        ],
        out_shape=[
            jax.ShapeDtypeStruct((_TOPK, _B), jnp.float32),
            jax.ShapeDtypeStruct((_ROWS // _RLANE, _RLANE), jnp.int32),
        ],
        scratch_shapes=[
            pltpu.VMEM((_B, _DK), jnp.float32),
            pltpu.VMEM((_N, _B), jnp.float32),
            pltpu.VMEM((_NBLK, _B, _DK), jnp.float32),
            pltpu.VMEM((_NBLK, _B, _DK), jnp.float32),
            pltpu.VMEM((_NBLK, _B, _DK), jnp.float32),
            pltpu.VMEM((_NBLK, _B, _DK), jnp.float32),
            pltpu.SemaphoreType.DMA,
            pltpu.SemaphoreType.DMA,
            pltpu.SemaphoreType.DMA,
            pltpu.SemaphoreType.DMA,
            pltpu.SemaphoreType.DMA,
        ],
    )(q, keys)


def _gather_body(rows_hbm, table_hbm, out_hbm, idx_v,
                 buf0, buf1, buf2, g0, g1, g2, s0, s1, s2):
    wid = lax.axis_index("s") * 2 + lax.axis_index("c")
    pltpu.sync_copy(rows_hbm.at[pl.ds(wid * _RPW, _RPW)], idx_v)
    bufs = (buf0, buf1, buf2)
    gsems = (g0, g1, g2)
    ssems = (s0, s1, s2)

    def gather(c):
        return pltpu.async_copy(
            table_hbm.at[idx_v.at[pl.ds(c * _CH, _CH)]],
            bufs[c % _NBUF], gsems[c % _NBUF])

    def scatter(c):
        return pltpu.async_copy(
            bufs[c % _NBUF],
            out_hbm.at[pl.ds(wid * _RPW + c * _CH, _CH)], ssems[c % _NBUF])

    handles = {}
    for c in range(min(_NBUF, _NCH)):
        handles[c] = gather(c)
    scat = {}
    for c in range(_NCH):
        handles[c].wait()
        scat[c] = scatter(c)
        nxt = c + _NBUF
        if nxt < _NCH:
            scat[c].wait()
            handles[nxt] = gather(nxt)
    for c in range(max(0, _NCH - _NBUF), _NCH):
        scat[c].wait()


def _gather_call(rows, table):
    mesh = plsc.VectorSubcoreMesh(core_axis_name="c", subcore_axis_name="s")
    f = functools.partial(
        pl.kernel,
        mesh=mesh,
        out_type=jax.ShapeDtypeStruct((_ROWS, _DV), jnp.float32),
        scratch_types=[
            pltpu.VMEM((_RPW,), jnp.int32),
            pltpu.VMEM((_CH, _DV), jnp.float32),
            pltpu.VMEM((_CH, _DV), jnp.float32),
            pltpu.VMEM((_CH, _DV), jnp.float32),
            pltpu.SemaphoreType.DMA,
            pltpu.SemaphoreType.DMA,
            pltpu.SemaphoreType.DMA,
            pltpu.SemaphoreType.DMA,
            pltpu.SemaphoreType.DMA,
            pltpu.SemaphoreType.DMA,
        ],
    )(_gather_body)
    return f(rows, table)


def kernel(query, keys, values):
    q = query.reshape(_B, _DK)
    w, rows = _topk_call(q, keys)
    rows = rows.reshape(_ROWS)
    table = values.reshape(_N * _L * _B, _DV)
    out = _gather_call(rows, table)
    topk_weights = jnp.transpose(w).reshape(_B, 1, _TOPK)
    outputs = jnp.swapaxes(out.reshape(_TOPK, _L, _B, _DV), 1, 2)
    return (topk_weights, outputs)


# speedup vs baseline: 1.0817x; 1.0013x over previous
import functools

import jax
import jax.numpy as jnp
from jax import lax
from jax.experimental import pallas as pl
from jax.experimental.pallas import tpu as pltpu
from jax.experimental.pallas import tpu_sc as plsc

_N = 128
_L = 20
_B = 64
_DK = 512
_DV = 512
_TOPK = 8

_NBLK = 32
_GRID = _N // _NBLK

_ROWS = _TOPK * _B * _L
_RLANE = 128
_NW = 32
_RPW = _ROWS // _NW
_CH = 80
_NCH = _RPW // _CH
_NBUF = 3


def _topk_body(q_hbm, keys_hbm, w_ref, rows_ref, q_v, s_ref,
               buf0, buf1, buf2, buf3, qsem, sem0, sem1, sem2, sem3):
    qcp = pltpu.make_async_copy(q_hbm, q_v, qsem)
    qcp.start()
    bufs = (buf0, buf1, buf2, buf3)
    sems = (sem0, sem1, sem2, sem3)

    def blk_copy(c):
        return pltpu.make_async_copy(
            keys_hbm.at[pl.ds(c * _NBLK, _NBLK)], bufs[c], sems[c])

    for c in range(_GRID):
        blk_copy(c).start()
    qcp.wait()
    q = q_v[...]
    for c in range(_GRID):
        blk_copy(c).wait()
        kb = bufs[c][...].reshape(_NBLK * _B, _DK)
        s_full = lax.dot_general(
            kb, q, (((1,), (1,)), ((), ())),
            precision=lax.Precision.DEFAULT,
            preferred_element_type=jnp.float32)
        s3 = s_full.reshape(_NBLK, _B, _B)
        diag = (lax.broadcasted_iota(jnp.int32, (_NBLK, _B, _B), 1) ==
                lax.broadcasted_iota(jnp.int32, (_NBLK, _B, _B), 2))
        s = jnp.sum(jnp.where(diag, s3, 0.0), axis=2)
        s_ref[pl.ds(c * _NBLK, _NBLK), :] = s / jnp.sqrt(jnp.float32(_DK))

    scores = s_ref[...]
    m = jnp.max(scores, axis=0, keepdims=True)
    e = jnp.exp(scores - m)
    att = e / jnp.sum(e, axis=0, keepdims=True)
    iota = lax.broadcasted_iota(jnp.int32, (_N, _B), 0)
    cur = att
    idxs = []
    vals = []
    for _ in range(_TOPK):
        mv = jnp.max(cur, axis=0, keepdims=True)
        mi = jnp.min(jnp.where(cur == mv, iota, _N), axis=0, keepdims=True)
        idxs.append(mi)
        vals.append(mv)
        cur = jnp.where(iota == mi, -1.0, cur)
    idx = jnp.concatenate(idxs, axis=0)
    w = jnp.concatenate(vals, axis=0)
    wm = jnp.max(w, axis=0, keepdims=True)
    we = jnp.exp(w - wm)
    w_ref[...] = we / jnp.sum(we, axis=0, keepdims=True)
    idx2 = jnp.concatenate([idx, idx], axis=1)
    idx3 = jnp.broadcast_to(idx2[:, None, :], (_TOPK, 10, 2 * _B))
    idxr = idx3.reshape(_ROWS // _RLANE, _RLANE)
    r_iota = lax.broadcasted_iota(jnp.int32, (_ROWS // _RLANE, _RLANE), 0)
    s_iota = lax.broadcasted_iota(jnp.int32, (_ROWS // _RLANE, _RLANE), 1)
    l_arr = (2 * r_iota + s_iota // _B) % _L
    b_arr = s_iota % _B
    rows_ref[...] = idxr * (_L * _B) + l_arr * _B + b_arr


def _topk_call(q, keys):
    return pl.pallas_call(
        _topk_body,
        in_specs=[
            pl.BlockSpec(memory_space=pltpu.MemorySpace.HBM),
            pl.BlockSpec(memory_space=pl.ANY),
        ],
        out_specs=[
            pl.BlockSpec((_TOPK, _B), lambda: (0, 0)),
            pl.BlockSpec((_ROWS // _RLANE, _RLANE), lambda: (0, 0)),
        ],
        out_shape=[
            jax.ShapeDtypeStruct((_TOPK, _B), jnp.float32),
            jax.ShapeDtypeStruct((_ROWS // _RLANE, _RLANE), jnp.int32),
        ],
        scratch_shapes=[
            pltpu.VMEM((_B, _DK), jnp.float32),
            pltpu.VMEM((_N, _B), jnp.float32),
            pltpu.VMEM((_NBLK, _B, _DK), jnp.float32),
            pltpu.VMEM((_NBLK, _B, _DK), jnp.float32),
            pltpu.VMEM((_NBLK, _B, _DK), jnp.float32),
            pltpu.VMEM((_NBLK, _B, _DK), jnp.float32),
            pltpu.SemaphoreType.DMA,
            pltpu.SemaphoreType.DMA,
            pltpu.SemaphoreType.DMA,
            pltpu.SemaphoreType.DMA,
            pltpu.SemaphoreType.DMA,
        ],
    )(q, keys)


def _gather_body(rows_hbm, table_hbm, out_hbm, idx_v,
                 buf0, buf1, buf2, g0, g1, g2, s0, s1, s2):
    wid = lax.axis_index("s") * 2 + lax.axis_index("c")
    pltpu.sync_copy(rows_hbm.at[pl.ds(wid * _RPW, _RPW)], idx_v)
    bufs = (buf0, buf1, buf2)
    gsems = (g0, g1, g2)
    ssems = (s0, s1, s2)

    def gather(c):
        return pltpu.async_copy(
            table_hbm.at[idx_v.at[pl.ds(c * _CH, _CH)]],
            bufs[c % _NBUF], gsems[c % _NBUF])

    def scatter(c):
        return pltpu.async_copy(
            bufs[c % _NBUF],
            out_hbm.at[pl.ds(wid * _RPW + c * _CH, _CH)], ssems[c % _NBUF])

    handles = {}
    for c in range(min(_NBUF, _NCH)):
        handles[c] = gather(c)
    scat = {}
    for c in range(_NCH):
        handles[c].wait()
        scat[c] = scatter(c)
        nxt = c + _NBUF
        if nxt < _NCH:
            scat[c].wait()
            handles[nxt] = gather(nxt)
    for c in range(max(0, _NCH - _NBUF), _NCH):
        scat[c].wait()


def _gather_call(rows, table):
    mesh = plsc.VectorSubcoreMesh(core_axis_name="c", subcore_axis_name="s")
    f = functools.partial(
        pl.kernel,
        mesh=mesh,
        out_type=jax.ShapeDtypeStruct((_ROWS, _DV), jnp.float32),
        scratch_types=[
            pltpu.VMEM((_RPW,), jnp.int32),
            pltpu.VMEM((_CH, _DV), jnp.float32),
            pltpu.VMEM((_CH, _DV), jnp.float32),
            pltpu.VMEM((_CH, _DV), jnp.float32),
            pltpu.SemaphoreType.DMA,
            pltpu.SemaphoreType.DMA,
            pltpu.SemaphoreType.DMA,
            pltpu.SemaphoreType.DMA,
            pltpu.SemaphoreType.DMA,
            pltpu.SemaphoreType.DMA,
        ],
    )(_gather_body)
    return f(rows, table)


def kernel(query, keys, values):
    q = query.reshape(_B, _DK)
    w, rows = _topk_call(q, keys)
    rows = rows.reshape(_ROWS)
    table = values.reshape(_N * _L * _B, _DV)
    out = _gather_call(rows, table)
    topk_weights = jnp.transpose(w).reshape(_B, 1, _TOPK)
    outputs = jnp.swapaxes(out.reshape(_TOPK, _L, _B, _DV), 1, 2)
    return (topk_weights, outputs)
